# baseline (speedup 1.0000x reference)
import functools

import jax
import jax.numpy as jnp
from jax import lax
from jax.experimental import pallas as pl
from jax.experimental.pallas import tpu as pltpu
from jax.experimental.pallas import tpu_sc as plsc

T_MAX = 4096
D_MODEL = 64
V_ROWS = 100000

NUM_CORES = 2
NUM_SUBCORES = 16
NW = NUM_CORES * NUM_SUBCORES
CHUNK = 400
NCHUNKS = V_ROWS // CHUNK
KMAX = -(-NCHUNKS // NW)
PE_W = T_MAX // NW


def _build_kernel():
    mesh = plsc.VectorSubcoreMesh(
        core_axis_name="c", subcore_axis_name="s",
        num_cores=NUM_CORES, num_subcores=NUM_SUBCORES)

    @functools.partial(
        pl.kernel,
        mesh=mesh,
        out_type=(
            jax.ShapeDtypeStruct((T_MAX, D_MODEL), jnp.float32),
            jax.ShapeDtypeStruct((V_ROWS, D_MODEL), jnp.float32),
        ),
    )
    def enc(pe_hbm, tab_hbm, pe_out, spat_out):
        wid = lax.axis_index("s") * NUM_CORES + lax.axis_index("c")

        pe_lo = wid * PE_W
        pltpu.sync_copy(pe_hbm.at[pl.ds(pe_lo, PE_W), :],
                        pe_out.at[pl.ds(pe_lo, PE_W), :])

        @pl.loop(0, KMAX)
        def _chunk(k):
            c = wid + k * NW
            @pl.when(c < NCHUNKS)
            def _():
                lo = c * CHUNK
                pltpu.sync_copy(tab_hbm.at[pl.ds(lo, CHUNK), :],
                                spat_out.at[pl.ds(lo, CHUNK), :])

    return enc


_ENC = None


def kernel(pos_encoding, spatial_table, T, V):
    global _ENC
    if _ENC is None:
        _ENC = _build_kernel()
    temporal, spatial = _ENC(pos_encoding, spatial_table)
    return temporal, spatial


# speedup vs baseline: 12.7442x; 12.7442x over previous
import functools

import jax
import jax.numpy as jnp
from jax import lax
from jax.experimental import pallas as pl
from jax.experimental.pallas import tpu as pltpu
from jax.experimental.pallas import tpu_sc as plsc

T_MAX = 4096
D_MODEL = 64
V_ROWS = 100000

NUM_CORES = 2
NUM_SUBCORES = 16
NW = NUM_CORES * NUM_SUBCORES
CHUNK = 400
NCHUNKS = V_ROWS // CHUNK
UNIFORM = NCHUNKS // NW
TAIL_W = NCHUNKS - UNIFORM * NW
KMAX = UNIFORM + 1
PE_W = T_MAX // NW


def _build_kernel():
    mesh = plsc.VectorSubcoreMesh(
        core_axis_name="c", subcore_axis_name="s",
        num_cores=NUM_CORES, num_subcores=NUM_SUBCORES)

    @functools.partial(
        pl.kernel,
        mesh=mesh,
        out_type=(
            jax.ShapeDtypeStruct((T_MAX, D_MODEL), jnp.float32),
            jax.ShapeDtypeStruct((V_ROWS, D_MODEL), jnp.float32),
        ),
        scratch_types=[
            pltpu.VMEM((2, CHUNK, D_MODEL), jnp.float32),
            pltpu.VMEM((PE_W, D_MODEL), jnp.float32),
            pltpu.SemaphoreType.DMA,
            pltpu.SemaphoreType.DMA,
            pltpu.SemaphoreType.DMA,
            pltpu.SemaphoreType.DMA,
            pltpu.SemaphoreType.DMA,
        ],
    )
    def enc(pe_hbm, tab_hbm, pe_out, spat_out,
            bufs, pe_v, rsem0, rsem1, wsem0, wsem1, psem):
        wid = lax.axis_index("s") * NUM_CORES + lax.axis_index("c")
        rsems = (rsem0, rsem1)
        wsems = (wsem0, wsem1)

        pe_lo = wid * PE_W
        pe_rd = pltpu.async_copy(pe_hbm.at[pl.ds(pe_lo, PE_W), :], pe_v, psem)

        def _guard(k, fn):
            if k < UNIFORM:
                fn()
            else:
                pl.when(wid < TAIL_W)(fn)

        rdesc, wdesc = {}, {}
        for k in range(KMAX):
            if k < UNIFORM:
                c = wid + k * NW
            else:
                c = jnp.minimum(UNIFORM * NW + wid, NCHUNKS - 1)
            lo = pl.multiple_of(c * CHUNK, 8)
            src = tab_hbm.at[pl.ds(lo, CHUNK), :]
            dst = spat_out.at[pl.ds(lo, CHUNK), :]
            rdesc[k] = pltpu.make_async_copy(src, bufs.at[k % 2], rsems[k % 2])
            wdesc[k] = pltpu.make_async_copy(bufs.at[k % 2], dst, wsems[k % 2])

        _guard(0, rdesc[0].start)
        for k in range(KMAX):
            _guard(k, rdesc[k].wait)
            _guard(k, wdesc[k].start)
            if k + 1 < KMAX:
                if k >= 1:
                    _guard(k - 1, wdesc[k - 1].wait)
                _guard(k + 1, rdesc[k + 1].start)
        pe_rd.wait()
        pltpu.sync_copy(pe_v, pe_out.at[pl.ds(pe_lo, PE_W), :])
        _guard(KMAX - 2, wdesc[KMAX - 2].wait)
        _guard(KMAX - 1, wdesc[KMAX - 1].wait)

    return enc


_ENC = None


def kernel(pos_encoding, spatial_table, T, V):
    global _ENC
    if _ENC is None:
        _ENC = _build_kernel()
    temporal, spatial = _ENC(pos_encoding, spatial_table)
    return temporal, spatial


# speedup vs baseline: 12.8857x; 1.0111x over previous
import functools

import jax
import jax.numpy as jnp
from jax import lax
from jax.experimental import pallas as pl
from jax.experimental.pallas import tpu as pltpu
from jax.experimental.pallas import tpu_sc as plsc

T_MAX = 4096
D_MODEL = 64
V_ROWS = 100000

NUM_CORES = 2
NUM_SUBCORES = 16
ROWS_SC = V_ROWS // NUM_CORES
ISS = 2
ROWS_ISS = ROWS_SC // ISS
CH = 3200
CSIZES = [CH] * (ROWS_ISS // CH) + (
    [ROWS_ISS % CH] if ROWS_ISS % CH else [])
PE_SC = T_MAX // NUM_CORES


def _build_kernel():
    mesh = plsc.VectorSubcoreMesh(
        core_axis_name="c", subcore_axis_name="s",
        num_cores=NUM_CORES, num_subcores=NUM_SUBCORES)

    @functools.partial(
        pl.kernel,
        mesh=mesh,
        out_type=(
            jax.ShapeDtypeStruct((T_MAX, D_MODEL), jnp.float32),
            jax.ShapeDtypeStruct((V_ROWS, D_MODEL), jnp.float32),
        ),
        scratch_types=[
            pltpu.VMEM_SHARED((ISS, 2, CH, D_MODEL), jnp.float32),
            pltpu.VMEM_SHARED((PE_SC, D_MODEL), jnp.float32),
            pltpu.SemaphoreType.DMA,
            pltpu.SemaphoreType.DMA,
            pltpu.SemaphoreType.DMA,
            pltpu.SemaphoreType.DMA,
        ],
    )
    def enc(pe_hbm, tab_hbm, pe_out, spat_out,
            bufs, pe_buf, rsem0, rsem1, wsem0, wsem1):
        c = lax.axis_index("c")
        s = lax.axis_index("s")
        rsems = (rsem0, rsem1)
        wsems = (wsem0, wsem1)

        for i in range(ISS):
            base = c * ROWS_SC + i * ROWS_ISS

            rdesc, wdesc = {}, {}
            off = 0
            for k, sz in enumerate(CSIZES):
                lo = pl.multiple_of(base + off, 8)
                src = tab_hbm.at[pl.ds(lo, sz), :]
                dst = spat_out.at[pl.ds(lo, sz), :]
                stage = bufs.at[i, k % 2, pl.ds(0, sz), :]
                rdesc[k] = pltpu.make_async_copy(src, stage, rsems[k % 2])
                wdesc[k] = pltpu.make_async_copy(stage, dst, wsems[k % 2])
                off += sz

            @pl.when(s == i)
            def _(rdesc=rdesc, wdesc=wdesc):
                nk = len(CSIZES)
                rdesc[0].start()
                for k in range(nk):
                    rdesc[k].wait()
                    wdesc[k].start()
                    if k + 1 < nk:
                        if k >= 1:
                            wdesc[k - 1].wait()
                        rdesc[k + 1].start()
                if nk >= 2:
                    wdesc[nk - 2].wait()
                wdesc[nk - 1].wait()

        @pl.when(s == ISS)
        def _():
            lo = pl.multiple_of(c * PE_SC, 8)
            pltpu.sync_copy(pe_hbm.at[pl.ds(lo, PE_SC), :], pe_buf)
            pltpu.sync_copy(pe_buf, pe_out.at[pl.ds(lo, PE_SC), :])

    return enc


_ENC = None


def kernel(pos_encoding, spatial_table, T, V):
    global _ENC
    if _ENC is None:
        _ENC = _build_kernel()
    temporal, spatial = _ENC(pos_encoding, spatial_table)
    return temporal, spatial


# speedup vs baseline: 12.9915x; 1.0082x over previous
import functools

import jax
import jax.numpy as jnp
from jax import lax
from jax.experimental import pallas as pl
from jax.experimental.pallas import tpu as pltpu
from jax.experimental.pallas import tpu_sc as plsc

T_MAX = 4096
D_MODEL = 64
V_ROWS = 100000

NUM_CORES = 2
NUM_SUBCORES = 16
NW = NUM_CORES * NUM_SUBCORES
CHUNK = 200
NBUF = 4
NCHUNKS = V_ROWS // CHUNK
UNIFORM = NCHUNKS // NW
TAIL_W = NCHUNKS - UNIFORM * NW
KMAX = UNIFORM + 1
PE_W = T_MAX // NW


def _build_kernel():
    mesh = plsc.VectorSubcoreMesh(
        core_axis_name="c", subcore_axis_name="s",
        num_cores=NUM_CORES, num_subcores=NUM_SUBCORES)

    @functools.partial(
        pl.kernel,
        mesh=mesh,
        out_type=(
            jax.ShapeDtypeStruct((T_MAX, D_MODEL), jnp.float32),
            jax.ShapeDtypeStruct((V_ROWS, D_MODEL), jnp.float32),
        ),
        scratch_types=[
            pltpu.VMEM((NBUF, CHUNK, D_MODEL), jnp.float32),
            pltpu.VMEM((PE_W, D_MODEL), jnp.float32),
        ] + [pltpu.SemaphoreType.DMA] * (2 * NBUF + 1),
    )
    def enc(pe_hbm, tab_hbm, pe_out, spat_out, bufs, pe_v, *sems):
        wid = lax.axis_index("s") * NUM_CORES + lax.axis_index("c")
        rsems = sems[:NBUF]
        wsems = sems[NBUF:2 * NBUF]
        psem = sems[2 * NBUF]

        pe_lo = wid * PE_W
        pe_rd = pltpu.make_async_copy(
            pe_hbm.at[pl.ds(pe_lo, PE_W), :], pe_v, psem)
        pe_wr = pltpu.make_async_copy(
            pe_v, pe_out.at[pl.ds(pe_lo, PE_W), :], psem)
        pe_rd.start()

        def _guard(k, fn):
            if k < UNIFORM:
                fn()
            else:
                pl.when(wid < TAIL_W)(fn)

        rdesc, wdesc = {}, {}
        for k in range(KMAX):
            if k < UNIFORM:
                c = wid + k * NW
            else:
                c = jnp.minimum(UNIFORM * NW + wid, NCHUNKS - 1)
            lo = pl.multiple_of(c * CHUNK, 8)
            src = tab_hbm.at[pl.ds(lo, CHUNK), :]
            dst = spat_out.at[pl.ds(lo, CHUNK), :]
            rdesc[k] = pltpu.make_async_copy(src, bufs.at[k % NBUF],
                                             rsems[k % NBUF])
            wdesc[k] = pltpu.make_async_copy(bufs.at[k % NBUF], dst,
                                             wsems[k % NBUF])

        for j in range(min(NBUF - 1, KMAX)):
            _guard(j, rdesc[j].start)
        for k in range(KMAX):
            _guard(k, rdesc[k].wait)
            _guard(k, wdesc[k].start)
            if k >= 1:
                _guard(k - 1, wdesc[k - 1].wait)
            j = k + NBUF - 1
            if j < KMAX:
                _guard(j, rdesc[j].start)
        pe_rd.wait()
        pe_wr.start()
        _guard(KMAX - 1, wdesc[KMAX - 1].wait)
        pe_wr.wait()

    return enc


_ENC = None


def kernel(pos_encoding, spatial_table, T, V):
    global _ENC
    if _ENC is None:
        _ENC = _build_kernel()
    temporal, spatial = _ENC(pos_encoding, spatial_table)
    return temporal, spatial
